# Initial kernel scaffold; baseline (speedup 1.0000x reference)
import jax
import jax.numpy as jnp
from jax.experimental import pallas as pl


def kernel(x, W_l1, W_r1, b1, p1, gamma, beta, W_l2, W_r2, b2, p2):
    raise NotImplementedError("write your pallas kernel here")


# speedup vs baseline: 7.3469x; 7.3469x over previous
import functools
import numpy as np
import jax
import jax.numpy as jnp
from jax.experimental import pallas as pl
from jax.experimental.pallas import tpu as pltpu

_B, _C, _H, _W = 4, 96, 32, 32
_OUT = 192
_K = 9
_N = _B * _H * _W
_N1 = _N // 4
_N2 = _N // 16
_BLK = 1024
_INF = np.float32(np.inf)
_BIG = np.float32(1e30)
_NINTH = np.float32(1.0) / np.float32(9.0)
_HIGH = jax.lax.Precision.HIGHEST
_VMEM_LIM = pltpu.CompilerParams(vmem_limit_bytes=60 * 1024 * 1024)


def _pos_enc_nodes():
    c, h, w = _C, _H, _W
    ch = int(np.ceil(c / 4) * 2)
    inv_freq = jnp.asarray(
        1.0 / (10000 ** (np.arange(0, ch, 2, dtype=np.float32) / ch)),
        dtype=jnp.float32)
    pos_x = jnp.arange(h, dtype=jnp.float32)
    pos_y = jnp.arange(w, dtype=jnp.float32)

    def get_emb(sin_inp):
        return jnp.stack([jnp.sin(sin_inp), jnp.cos(sin_inp)],
                         axis=-1).reshape(sin_inp.shape[0], -1)

    emb_x = get_emb(pos_x[:, None] * inv_freq[None, :])[:, None, :]
    emb_y = get_emb(pos_y[:, None] * inv_freq[None, :])[None, :, :]
    emb = jnp.zeros((h, w, 2 * ch), dtype=jnp.float32)
    emb = emb.at[:, :, :ch].set(jnp.broadcast_to(emb_x, (h, w, ch)))
    emb = emb.at[:, :, ch:2 * ch].set(jnp.broadcast_to(emb_y, (h, w, ch)))
    pe = emb[:, :, :c].reshape(h * w, c)
    return jnp.tile(pe, (_B, 1))


def _rowsum_sq(x):
    x2 = x * x
    p = x2[:, 0:8]
    for t in range(1, 12):
        p = p + x2[:, 8 * t:8 * t + 8]
    a = p[:, 0:4] + p[:, 4:8]
    b = a[:, 0:2] + a[:, 2:4]
    return b[:, 0:1] + b[:, 1:2]


def _mean9_tree(v):
    a0 = v[0] + v[8]
    b0 = a0 + v[4]
    b1 = v[1] + v[5]
    b2 = v[2] + v[6]
    b3 = v[3] + v[7]
    c0 = b0 + b2
    c1 = b1 + b3
    return (c0 + c1) * _NINTH


def _knn_mean_agg(dist, xb, scr_ref):
    r, cn = dist.shape
    col = jax.lax.broadcasted_iota(jnp.int32, (r, cn), 1)
    work = dist
    for t in range(_K):
        m = jnp.min(work, axis=1, keepdims=True)
        j = jnp.min(jnp.where(work == m, col, cn), axis=1, keepdims=True)
        onehot = jnp.where(col == j, np.float32(1.0), np.float32(0.0))
        scr_ref[t] = jnp.dot(onehot, xb, preferred_element_type=jnp.float32,
                             precision=_HIGH)
        work = jnp.where(col == j, _INF, work)
    return _mean9_tree([scr_ref[t] for t in range(_K)])


def _stable_rank(s, chunk):
    n = s.shape[0]
    nch = n // chunk
    ranks = []
    for ci in range(nch):
        si = s[ci * chunk:(ci + 1) * chunk]
        acc = jnp.zeros((chunk, 1), dtype=jnp.float32)
        for cj in range(nch):
            sj = s[cj * chunk:(cj + 1) * chunk]
            ii = jax.lax.broadcasted_iota(jnp.int32, (chunk, chunk), 0) \
                + ci * chunk
            jj = jax.lax.broadcasted_iota(jnp.int32, (chunk, chunk), 1) \
                + cj * chunk
            cmp = (sj.T > si) | ((sj.T == si) & (jj < ii))
            acc = acc + jnp.sum(
                jnp.where(cmp, np.float32(1.0), np.float32(0.0)),
                axis=1, keepdims=True)
        ranks.append(acc)
    return jnp.concatenate(ranks, axis=0).astype(jnp.int32)


def _pool_gather(rank, feats, kout, chunk):
    n = rank.shape[0]
    nch = n // chunk
    riota = jax.lax.broadcasted_iota(jnp.int32, (kout, chunk), 0)
    outs = [jnp.zeros((kout, f.shape[1]), dtype=jnp.float32)
            for f in feats]
    for cj in range(nch):
        rj = rank[cj * chunk:(cj + 1) * chunk]
        pc = jnp.where(rj.T == riota, np.float32(1.0), np.float32(0.0))
        for t, f in enumerate(feats):
            fj = f[cj * chunk:(cj + 1) * chunk]
            outs[t] = outs[t] + jnp.dot(
                pc, fj, preferred_element_type=jnp.float32,
                precision=_HIGH)
    return outs


def _knn1_kernel(xb_ref, n8_ref, agg_ref, scr_ref):
    pid = pl.program_id(0)
    xb = xb_ref[...]
    d2 = _rowsum_sq(xb)
    g = jnp.dot(xb, xb.T, preferred_element_type=jnp.float32)
    dist = d2 + d2.T - 2.0 * g
    row = jax.lax.broadcasted_iota(jnp.int32, (_BLK, _BLK), 0)
    colx = jax.lax.broadcasted_iota(jnp.int32, (_BLK, _BLK), 1)
    last = (pid == 3)
    cross = last & ((row == _BLK - 1) ^ (colx == _BLK - 1))
    dist = jnp.where(cross, _BIG, dist)
    agg_b = _knn_mean_agg(dist, xb, scr_ref)
    n8 = n8_ref[...]
    special = _mean9_tree(
        [xb[_BLK - 1:_BLK]] + [n8[t:t + 1] for t in range(8)])
    row1 = jax.lax.broadcasted_iota(jnp.int32, (_BLK, 1), 0)
    agg_ref[...] = jnp.where(last & (row1 == _BLK - 1), special, agg_b)


def _conv_pool1_kernel(nodes_ref, agg_ref, bf_ref,
                       wl1_ref, wr1_ref, b1_ref, p1_ref, p1r_ref,
                       gt_ref, beta_ref,
                       h1g_ref, b1p_ref):
    nodes = nodes_ref[...]
    h1 = (jnp.dot(agg_ref[...], wl1_ref[...],
                  preferred_element_type=jnp.float32)
          + jnp.dot(nodes, wr1_ref[...], preferred_element_type=jnp.float32)
          + b1_ref[...])
    nrm1 = jnp.sqrt(_rowsum_sq(p1r_ref[...])[0, 0]) + 1e-16
    s1 = jnp.tanh(
        jnp.dot(h1, p1_ref[...], preferred_element_type=jnp.float32)
        / nrm1)
    rank1 = _stable_rank(s1, _BLK)
    h1p, s1p, b1p = _pool_gather(rank1, [h1, s1, bf_ref[...]], _N1, _BLK)
    h1p = h1p * s1p
    sq = h1p * h1p
    norm = (jnp.dot(sq, gt_ref[...], preferred_element_type=jnp.float32)
            + beta_ref[...])
    h1g_ref[...] = h1p * jax.lax.rsqrt(norm)
    b1p_ref[...] = b1p


def _knn2_kernel(h1g_ref, b1p_ref, wl2_ref, wr2_ref, b2_ref, h2_ref,
                 scr_ref):
    h1g = h1g_ref[...]
    b1p = b1p_ref[...]
    d2b = _rowsum_sq(h1g)
    g2 = jnp.dot(h1g, h1g.T, preferred_element_type=jnp.float32)
    dist2 = d2b + d2b.T - 2.0 * g2
    dist2 = jnp.where(b1p != b1p.T, _BIG, dist2)
    agg2 = _knn_mean_agg(dist2, h1g, scr_ref)
    h2_ref[...] = (
        jnp.dot(agg2, wl2_ref[...], preferred_element_type=jnp.float32)
        + jnp.dot(h1g, wr2_ref[...], preferred_element_type=jnp.float32)
        + b2_ref[...])


def _pool2_kernel(h2_ref, nrm_ref, p2_ref, out_ref):
    h2 = h2_ref[...]
    s2 = jnp.tanh(
        jnp.dot(h2, p2_ref[...], preferred_element_type=jnp.float32)
        / nrm_ref[0, 0])
    rank2 = _stable_rank(s2, _N1)
    h2p, s2p = _pool_gather(rank2, [h2, s2], _N2, _N1)
    out_ref[...] = h2p * s2p


def kernel(x, W_l1, W_r1, b1, p1, gamma, beta, W_l2, W_r2, b2, p2):
    xt = jnp.transpose(x, (0, 2, 3, 1)).reshape(_N, _C)
    pe = _pos_enc_nodes()
    batch_f = jnp.floor(
        jnp.linspace(0.0, float(_B), _N)).astype(jnp.int32).astype(
        jnp.float32).reshape(_N, 1)
    nrm2 = (jnp.linalg.norm(p2) + 1e-16).reshape(1, 1)

    nodes = xt + pe

    blk_spec = pl.BlockSpec((_BLK, _C), lambda i: (i, 0))
    row8_spec = pl.BlockSpec((8, _C), lambda i: (0, 0))
    agg = pl.pallas_call(
        _knn1_kernel,
        grid=(4,),
        in_specs=[blk_spec, row8_spec],
        out_specs=blk_spec,
        out_shape=jax.ShapeDtypeStruct((_N, _C), jnp.float32),
        scratch_shapes=[pltpu.VMEM((_K, _BLK, _C), jnp.float32)],
        compiler_params=_VMEM_LIM,
    )(nodes, nodes[0:8])

    h1g, b1p = pl.pallas_call(
        _conv_pool1_kernel,
        out_shape=[jax.ShapeDtypeStruct((_N1, _C), jnp.float32),
                   jax.ShapeDtypeStruct((_N1, 1), jnp.float32)],
        compiler_params=_VMEM_LIM,
    )(nodes, agg, batch_f, W_l1, W_r1, b1.reshape(1, _C),
      p1.reshape(_C, 1), p1.reshape(1, _C), gamma.T, beta.reshape(1, _C))

    h2 = pl.pallas_call(
        _knn2_kernel,
        out_shape=jax.ShapeDtypeStruct((_N1, _OUT), jnp.float32),
        scratch_shapes=[pltpu.VMEM((_K, _BLK, _C), jnp.float32)],
        compiler_params=_VMEM_LIM,
    )(h1g, b1p, W_l2, W_r2, b2.reshape(1, _OUT))

    out = pl.pallas_call(
        _pool2_kernel,
        out_shape=jax.ShapeDtypeStruct((_N2, _OUT), jnp.float32),
        compiler_params=_VMEM_LIM,
    )(h2, nrm2, p2.reshape(_OUT, 1))

    return jnp.transpose(out.reshape(_B, _H // 4, _W // 4, _OUT),
                         (0, 3, 1, 2))
